# Initial kernel scaffold; baseline (speedup 1.0000x reference)
import jax
import jax.numpy as jnp
from jax.experimental import pallas as pl


def kernel(features, edge_index, W, b):
    raise NotImplementedError("write your pallas kernel here")


# speedup vs baseline: 5.5117x; 5.5117x over previous
import functools

import jax
import jax.numpy as jnp
from jax import lax
from jax.experimental import pallas as pl
from jax.experimental.pallas import tpu as pltpu
from jax.experimental.pallas import tpu_sc as plsc

N_NODES = 10000
N_EDGES = 320000
D = 128

NC = 2
NS = 16
NW = NC * NS

EDGES_PER_WORKER = N_EDGES // NW
CHUNK = 80
CHUNKS = EDGES_PER_WORKER // CHUNK
ROWS_MAIN = 640
ROWS_LAST = N_NODES - 15 * ROWS_MAIN


def _sc_scatter_body(src_hbm, dst_hbm, feat_hbm, part_hbm,
                     idx_src, idx_dst, rows, acc, sem):
    c = lax.axis_index("c")
    s = lax.axis_index("s")
    wid = s * NC + c

    @pl.when(s < 15)
    def _():
        pltpu.sync_copy(feat_hbm.at[pl.ds(s * ROWS_MAIN, ROWS_MAIN)],
                        acc.at[pl.ds(s * ROWS_MAIN, ROWS_MAIN)])

    @pl.when(s == 15)
    def _():
        pltpu.sync_copy(feat_hbm.at[pl.ds(15 * ROWS_MAIN, ROWS_LAST)],
                        acc.at[pl.ds(15 * ROWS_MAIN, ROWS_LAST)])

    plsc.subcore_barrier()

    base = wid * EDGES_PER_WORKER

    def body(i, carry):
        off = base + i * CHUNK
        pltpu.sync_copy(src_hbm.at[pl.ds(off, CHUNK)], idx_src)
        pltpu.sync_copy(dst_hbm.at[pl.ds(off, CHUNK)], idx_dst)
        pltpu.async_copy(feat_hbm.at[idx_src], rows, sem).wait()
        pltpu.sync_copy(rows, acc.at[idx_dst], add=True)
        return carry

    lax.fori_loop(0, CHUNKS, body, 0)
    plsc.subcore_barrier()

    @pl.when(s < 15)
    def _():
        pltpu.sync_copy(acc.at[pl.ds(s * ROWS_MAIN, ROWS_MAIN)],
                        part_hbm.at[c, pl.ds(s * ROWS_MAIN, ROWS_MAIN)])

    @pl.when(s == 15)
    def _():
        pltpu.sync_copy(acc.at[pl.ds(15 * ROWS_MAIN, ROWS_LAST)],
                        part_hbm.at[c, pl.ds(15 * ROWS_MAIN, ROWS_LAST)])


@functools.partial(jax.jit, static_argnums=())
def _sc_scatter(src, dst, features):
    mesh = plsc.VectorSubcoreMesh(core_axis_name="c", subcore_axis_name="s")
    f = pl.kernel(
        _sc_scatter_body,
        out_type=jax.ShapeDtypeStruct((NC, N_NODES, D), jnp.float32),
        mesh=mesh,
        scratch_types=[
            pltpu.VMEM((CHUNK,), jnp.int32),
            pltpu.VMEM((CHUNK,), jnp.int32),
            pltpu.VMEM((CHUNK, D), jnp.float32),
            pltpu.VMEM_SHARED((N_NODES, D), jnp.float32),
            pltpu.SemaphoreType.DMA,
        ],
    )
    return f(src, dst, features)


def _tc_finish_body(p_ref, f_ref, w_ref, b_ref, o_ref):
    h = p_ref[0] + p_ref[1] - f_ref[...]
    norm = jnp.sqrt(jnp.sum(h * h, axis=1, keepdims=True))
    hn = h / jnp.maximum(norm, 1e-12)
    o_ref[...] = lax.dot_general(
        hn, w_ref[...], (((1,), (1,)), ((), ())),
        preferred_element_type=jnp.float32) + b_ref[...]


def _tc_finish(parts, features, W, b2d):
    R = 1000
    grid = N_NODES // R
    return pl.pallas_call(
        _tc_finish_body,
        grid=(grid,),
        in_specs=[
            pl.BlockSpec((NC, R, D), lambda i: (0, i, 0)),
            pl.BlockSpec((R, D), lambda i: (i, 0)),
            pl.BlockSpec((D, D), lambda i: (0, 0)),
            pl.BlockSpec((1, D), lambda i: (0, 0)),
        ],
        out_specs=pl.BlockSpec((R, D), lambda i: (i, 0)),
        out_shape=jax.ShapeDtypeStruct((N_NODES, D), jnp.float32),
    )(parts, features, W, b2d)


def kernel(features, edge_index, W, b):
    src = edge_index[0].astype(jnp.int32)
    dst = edge_index[1].astype(jnp.int32)
    parts = _sc_scatter(src, dst, features)
    return _tc_finish(parts, features, W, b.reshape(1, D))


# speedup vs baseline: 9.7790x; 1.7742x over previous
import functools

import jax
import jax.numpy as jnp
from jax import lax
from jax.experimental import pallas as pl
from jax.experimental.pallas import tpu as pltpu
from jax.experimental.pallas import tpu_sc as plsc

N_NODES = 10000
N_EDGES = 320000
D = 128

NC = 2
NS = 16
NW = NC * NS

EDGES_PER_WORKER = N_EDGES // NW
CHUNK = 80
CHUNKS = EDGES_PER_WORKER // CHUNK
ROWS_MAIN = 640
ROWS_LAST = N_NODES - 15 * ROWS_MAIN


def _sc_scatter_body(src_hbm, dst_hbm, feat_hbm, part_hbm,
                     isrc0, idst0, rows0, isrc1, idst1, rows1, acc,
                     sem_i0, sem_g0, sem_s0, sem_i1, sem_g1, sem_s1):
    c = lax.axis_index("c")
    s = lax.axis_index("s")
    wid = s * NC + c

    @pl.when(s < 15)
    def _():
        pltpu.sync_copy(feat_hbm.at[pl.ds(s * ROWS_MAIN, ROWS_MAIN)],
                        acc.at[pl.ds(s * ROWS_MAIN, ROWS_MAIN)])

    @pl.when(s == 15)
    def _():
        pltpu.sync_copy(feat_hbm.at[pl.ds(15 * ROWS_MAIN, ROWS_LAST)],
                        acc.at[pl.ds(15 * ROWS_MAIN, ROWS_LAST)])

    plsc.subcore_barrier()

    base = wid * EDGES_PER_WORKER
    bufs = ((isrc0, idst0, rows0, sem_i0, sem_g0, sem_s0),
            (isrc1, idst1, rows1, sem_i1, sem_g1, sem_s1))

    def start_idx(i, b):
        isrc, idst, _, sem_i, _, _ = b
        off = base + i * CHUNK
        pltpu.async_copy(src_hbm.at[pl.ds(off, CHUNK)], isrc, sem_i)
        pltpu.async_copy(dst_hbm.at[pl.ds(off, CHUNK)], idst, sem_i)

    def wait_idx(b):
        isrc, idst, _, sem_i, _, _ = b
        pltpu.make_async_copy(src_hbm.at[pl.ds(0, CHUNK)], isrc, sem_i).wait()
        pltpu.make_async_copy(dst_hbm.at[pl.ds(0, CHUNK)], idst, sem_i).wait()

    def start_gather(b):
        isrc, _, rows, _, sem_g, _ = b
        pltpu.async_copy(feat_hbm.at[isrc], rows, sem_g)

    def wait_gather(b):
        isrc, _, rows, _, sem_g, _ = b
        pltpu.make_async_copy(feat_hbm.at[isrc], rows, sem_g).wait()

    def start_scatter(b):
        _, idst, rows, _, _, sem_s = b
        pltpu.async_copy(rows, acc.at[idst], sem_s, add=True)

    def wait_scatter(b):
        _, idst, rows, _, _, sem_s = b
        pltpu.make_async_copy(rows, acc.at[idst], sem_s).wait()

    def step(i, cur, nxt):
        @pl.when(i >= 1)
        def _():
            wait_scatter(nxt)

        @pl.when(i + 1 < CHUNKS)
        def _():
            start_idx(i + 1, nxt)

        wait_gather(cur)
        start_scatter(cur)

        @pl.when(i + 1 < CHUNKS)
        def _():
            wait_idx(nxt)
            start_gather(nxt)

    start_idx(0, bufs[0])
    wait_idx(bufs[0])
    start_gather(bufs[0])

    def body(i, carry):
        @pl.when(i % 2 == 0)
        def _():
            step(i, bufs[0], bufs[1])

        @pl.when(i % 2 == 1)
        def _():
            step(i, bufs[1], bufs[0])

        return carry

    lax.fori_loop(0, CHUNKS, body, 0)
    wait_scatter(bufs[(CHUNKS - 1) % 2])
    plsc.subcore_barrier()

    @pl.when(s < 15)
    def _():
        pltpu.sync_copy(acc.at[pl.ds(s * ROWS_MAIN, ROWS_MAIN)],
                        part_hbm.at[c, pl.ds(s * ROWS_MAIN, ROWS_MAIN)])

    @pl.when(s == 15)
    def _():
        pltpu.sync_copy(acc.at[pl.ds(15 * ROWS_MAIN, ROWS_LAST)],
                        part_hbm.at[c, pl.ds(15 * ROWS_MAIN, ROWS_LAST)])


@functools.partial(jax.jit, static_argnums=())
def _sc_scatter(src, dst, features):
    mesh = plsc.VectorSubcoreMesh(core_axis_name="c", subcore_axis_name="s")
    f = pl.kernel(
        _sc_scatter_body,
        out_type=jax.ShapeDtypeStruct((NC, N_NODES, D), jnp.float32),
        mesh=mesh,
        scratch_types=[
            pltpu.VMEM((CHUNK,), jnp.int32),
            pltpu.VMEM((CHUNK,), jnp.int32),
            pltpu.VMEM((CHUNK, D), jnp.float32),
            pltpu.VMEM((CHUNK,), jnp.int32),
            pltpu.VMEM((CHUNK,), jnp.int32),
            pltpu.VMEM((CHUNK, D), jnp.float32),
            pltpu.VMEM_SHARED((N_NODES, D), jnp.float32),
            pltpu.SemaphoreType.DMA,
            pltpu.SemaphoreType.DMA,
            pltpu.SemaphoreType.DMA,
            pltpu.SemaphoreType.DMA,
            pltpu.SemaphoreType.DMA,
            pltpu.SemaphoreType.DMA,
        ],
    )
    return f(src, dst, features)


def _tc_finish_body(p_ref, f_ref, w_ref, b_ref, o_ref):
    h = p_ref[0] + p_ref[1] - f_ref[...]
    norm = jnp.sqrt(jnp.sum(h * h, axis=1, keepdims=True))
    hn = h / jnp.maximum(norm, 1e-12)
    o_ref[...] = lax.dot_general(
        hn, w_ref[...], (((1,), (1,)), ((), ())),
        preferred_element_type=jnp.float32) + b_ref[...]


def _tc_finish(parts, features, W, b2d):
    R = 1000
    grid = N_NODES // R
    return pl.pallas_call(
        _tc_finish_body,
        grid=(grid,),
        in_specs=[
            pl.BlockSpec((NC, R, D), lambda i: (0, i, 0)),
            pl.BlockSpec((R, D), lambda i: (i, 0)),
            pl.BlockSpec((D, D), lambda i: (0, 0)),
            pl.BlockSpec((1, D), lambda i: (0, 0)),
        ],
        out_specs=pl.BlockSpec((R, D), lambda i: (i, 0)),
        out_shape=jax.ShapeDtypeStruct((N_NODES, D), jnp.float32),
    )(parts, features, W, b2d)


def kernel(features, edge_index, W, b):
    src = edge_index[0].astype(jnp.int32)
    dst = edge_index[1].astype(jnp.int32)
    parts = _sc_scatter(src, dst, features)
    return _tc_finish(parts, features, W, b.reshape(1, D))
